# Initial kernel scaffold; baseline (speedup 1.0000x reference)
import jax
import jax.numpy as jnp
from jax.experimental import pallas as pl


def kernel(input_ids, table, W_score):
    raise NotImplementedError("write your pallas kernel here")


# speedup vs baseline: 1.1320x; 1.1320x over previous
import functools

import jax
import jax.numpy as jnp
from jax import lax
from jax.experimental import pallas as pl
from jax.experimental.pallas import tpu as pltpu
from jax.experimental.pallas import tpu_sc as plsc

_VOCAB = 50257
_D = 1024
_SEQ = 4096
_NLAYERS = 24
_NSKIP = 4

_NC = 2
_NS = 16
_NW = _NC * _NS
_ROWS_PER_W = _SEQ // _NW
_CH = 64


def _sc_gather_body(ids_hbm, table_hbm, hid_hbm, idx_a, idx_b, rows_v, sem):
    c = lax.axis_index("c")
    s = lax.axis_index("s")
    wid = s * _NC + c
    base = wid * _ROWS_PER_W
    pltpu.sync_copy(ids_hbm.at[pl.ds(base, _CH)], idx_a)
    pltpu.sync_copy(ids_hbm.at[pl.ds(base + _CH, _CH)], idx_b)
    pltpu.async_copy(table_hbm.at[idx_a], rows_v, sem).wait()
    pltpu.sync_copy(rows_v, hid_hbm.at[pl.ds(base, _CH)])
    pltpu.async_copy(table_hbm.at[idx_b], rows_v, sem).wait()
    pltpu.sync_copy(rows_v, hid_hbm.at[pl.ds(base + _CH, _CH)])


@functools.cache
def _sc_gather():
    return pl.kernel(
        _sc_gather_body,
        mesh=plsc.VectorSubcoreMesh(core_axis_name="c", subcore_axis_name="s"),
        out_type=jax.ShapeDtypeStruct((_SEQ, _D), jnp.float32),
        scratch_types=[
            pltpu.VMEM((_CH,), jnp.int32),
            pltpu.VMEM((_CH,), jnp.int32),
            pltpu.VMEM((_CH, _D), jnp.float32),
            pltpu.SemaphoreType.DMA,
        ],
    )


def _score_body(hid_ref, w_ref, out_ref):
    pooled = jnp.sum(hid_ref[...], axis=0, keepdims=True) * (1.0 / _SEQ)
    scores = jnp.dot(pooled, w_ref[...], preferred_element_type=jnp.float32,
                     precision=lax.Precision.HIGHEST)
    a = jnp.abs(scores)
    idxs = lax.broadcasted_iota(jnp.int32, (1, _NLAYERS), 1)
    for k in range(_NSKIP):
        m = jnp.min(a)
        i = jnp.min(jnp.where(a <= m, idxs, jnp.int32(2**30)))
        out_ref[k] = i
        a = jnp.where(idxs == i, jnp.float32(jnp.inf), a)


def _score_topk(hid, w):
    return pl.pallas_call(
        _score_body,
        out_shape=jax.ShapeDtypeStruct((_NSKIP,), jnp.int32),
        out_specs=pl.BlockSpec(memory_space=pltpu.SMEM),
    )(hid, w)


def kernel(input_ids, table, W_score):
    ids = input_ids.reshape(_SEQ).astype(jnp.int32)
    hid = _sc_gather()(ids, table)
    skip = _score_topk(hid, W_score)
    return hid.reshape(1, _SEQ, _D), skip
